# baseline (speedup 1.0000x reference)
import functools

import jax
import jax.numpy as jnp
from jax import lax
from jax.experimental import pallas as pl
from jax.experimental.pallas import tpu as pltpu
from jax.experimental.pallas import tpu_sc as plsc

_NC = 2
_NS = 16
_L = 16


def _round_up(x, m):
    return (x + m - 1) // m * m


def _make_sc_hist(e_pad, np_rows):
    per_tile = e_pad // _NS
    n_vec = per_tile // _L
    seg = np_rows // _NS

    mesh = plsc.VectorSubcoreMesh(core_axis_name="c", subcore_axis_name="s")

    @functools.partial(
        pl.kernel,
        out_type=(
            jax.ShapeDtypeStruct((np_rows,), jnp.float32),
            jax.ShapeDtypeStruct((np_rows,), jnp.float32),
        ),
        mesh=mesh,
        scratch_types=[
            pltpu.VMEM((per_tile,), jnp.int32),
            pltpu.VMEM((np_rows,), jnp.float32),
            pltpu.VMEM((seg,), jnp.float32),
            pltpu.VMEM((seg,), jnp.float32),
            pltpu.VMEM_SHARED((_NS, np_rows), jnp.float32),
        ],
        compiler_params=pltpu.CompilerParams(needs_layout_passes=False),
    )
    def hist_kernel(src_hbm, dst_hbm, out_src, out_dst,
                    idx_v, hist_v, acc_v, seg_v, shared):
        cid = lax.axis_index("c")
        sid = lax.axis_index("s")
        zeros = jnp.zeros((_L,), jnp.float32)
        ones = jnp.ones((_L,), jnp.float32)

        def zero_body(i, _):
            hist_v[pl.ds(i * _L, _L)] = zeros
            return 0
        lax.fori_loop(0, np_rows // _L, zero_body, 0)

        def count_from(ref):
            pltpu.sync_copy(ref.at[pl.ds(sid * per_tile, per_tile)], idx_v)

            def body(k, _):
                iv = idx_v[pl.ds(k * _L, _L)]
                plsc.addupdate_scatter(hist_v, [iv], ones)
                return 0
            lax.fori_loop(0, n_vec, body, 0)

        @pl.when(cid == 0)
        def _():
            count_from(src_hbm)

        @pl.when(cid == 1)
        def _():
            count_from(dst_hbm)

        pltpu.sync_copy(hist_v, shared.at[sid])
        plsc.subcore_barrier()
        pltpu.sync_copy(shared.at[0, pl.ds(sid * seg, seg)], acc_v)

        def red_body(t, _):
            pltpu.sync_copy(shared.at[t, pl.ds(sid * seg, seg)], seg_v)

            def add_body(j, _):
                sl = pl.ds(j * _L, _L)
                acc_v[sl] = acc_v[sl] + seg_v[sl]
                return 0
            lax.fori_loop(0, seg // _L, add_body, 0)
            return 0
        lax.fori_loop(1, _NS, red_body, 0)

        @pl.when(cid == 0)
        def _():
            pltpu.sync_copy(acc_v, out_src.at[pl.ds(sid * seg, seg)])

        @pl.when(cid == 1)
        def _():
            pltpu.sync_copy(acc_v, out_dst.at[pl.ds(sid * seg, seg)])

    return hist_kernel


def _make_sc_scatter(e_pad, np_acc, fs):
    chunk = 2048
    nb = chunk // _L
    npair = e_pad // (2 * chunk)

    mesh = plsc.VectorSubcoreMesh(core_axis_name="c", subcore_axis_name="s")

    nflat = np_acc * fs
    ut = jax.ShapeDtypeStruct((_NS, nflat), jnp.float32)

    @functools.partial(
        pl.kernel,
        out_type=(ut, ut, ut, ut),
        mesh=mesh,
        scratch_types=[
            pltpu.VMEM((chunk,), jnp.int32),
            pltpu.VMEM((chunk,), jnp.int32),
            pltpu.VMEM((chunk,), jnp.int32),
            pltpu.VMEM((chunk,), jnp.int32),
            pltpu.VMEM((nflat,), jnp.float32),
            pltpu.VMEM((nflat,), jnp.float32),
            pltpu.SemaphoreType.DMA,
            pltpu.SemaphoreType.DMA,
        ],
        compiler_params=pltpu.CompilerParams(needs_layout_passes=False),
    )
    def scatter_kernel(src_hbm, dst_hbm, u0a, u0b, u1a, u1b,
                       out0a, out0b, out1a, out1b,
                       sidx_a, didx_a, sidx_b, didx_b, u_sl, acc_sl,
                       sem_a, sem_b):
        cid = lax.axis_index("c")
        sid = lax.axis_index("s")
        def process(sidx, didx):
            def batch(k, _):
                s16 = sidx[pl.ds(k * _L, _L)] * fs
                d16 = didx[pl.ds(k * _L, _L)] * fs
                for f in range(fs):
                    vals = plsc.load_gather(u_sl, [s16 + f])
                    plsc.addupdate_scatter(acc_sl, [d16 + f], vals)
                return 0
            lax.fori_loop(0, nb, batch, 0)

        def load_a(g):
            return (pltpu.async_copy(
                        src_hbm.at[pl.ds(g * chunk, chunk)], sidx_a, sem_a),
                    pltpu.async_copy(
                        dst_hbm.at[pl.ds(g * chunk, chunk)], didx_a, sem_a))

        def run_pass(u_in, out_hbm):
            pltpu.sync_copy(u_in.at[sid], u_sl)
            pltpu.sync_copy(u_in.at[sid], acc_sl)

            pltpu.sync_copy(src_hbm.at[pl.ds(0, chunk)], sidx_a)
            pltpu.sync_copy(dst_hbm.at[pl.ds(0, chunk)], didx_a)

            def pair(g, _):
                base_b = (2 * g + 1) * chunk
                db0 = pltpu.async_copy(
                    src_hbm.at[pl.ds(base_b, chunk)], sidx_b, sem_b)
                db1 = pltpu.async_copy(
                    dst_hbm.at[pl.ds(base_b, chunk)], didx_b, sem_b)
                process(sidx_a, didx_a)
                db0.wait()
                db1.wait()

                @pl.when(g + 1 < npair)
                def _():
                    load_a(2 * g + 2)
                process(sidx_b, didx_b)

                @pl.when(g + 1 < npair)
                def _():
                    pltpu.make_async_copy(
                        src_hbm.at[pl.ds(0, chunk)], sidx_a, sem_a).wait()
                    pltpu.make_async_copy(
                        dst_hbm.at[pl.ds(0, chunk)], didx_a, sem_a).wait()
                return 0
            lax.fori_loop(0, npair, pair, 0)

            pltpu.sync_copy(acc_sl, out_hbm.at[sid])

        @pl.when(cid == 0)
        def _():
            run_pass(u0a, out0a)
            run_pass(u0b, out0b)

        @pl.when(cid == 1)
        def _():
            run_pass(u1a, out1a)
            run_pass(u1b, out1b)

    return scatter_kernel


def _make_tc_matmul(n_nodes, m_mid, d, np_rows, br):
    grid = (np_rows // br,)
    h = d // 2

    def body(ci_ref, ch_ref, nh_ref, convW_ref, fusW_ref, hs_ref,
             u0a_ref, u0b_ref, u1a_ref, u1b_ref):
        ns = lax.rsqrt(hs_ref[...] + 1.0)
        ch = ch_ref[...] * ns
        z0 = jnp.dot(ch, convW_ref[...], preferred_element_type=jnp.float32)
        u0a_ref[...] = z0[:, :h]
        u0b_ref[...] = z0[:, h:]
        fused = jnp.dot(ci_ref[...], nh_ref[...],
                        preferred_element_type=jnp.float32) * ns
        z1 = jnp.dot(fused, fusW_ref[...],
                     preferred_element_type=jnp.float32)
        u1a_ref[...] = z1[:, :h]
        u1b_ref[...] = z1[:, h:]

    qspec = pl.BlockSpec((br, h), lambda i: (i, 0))
    qshape = jax.ShapeDtypeStruct((np_rows, h), jnp.float32)
    return pl.pallas_call(
        body,
        grid=grid,
        in_specs=[
            pl.BlockSpec((br, m_mid), lambda i: (i, 0)),
            pl.BlockSpec((br, d), lambda i: (i, 0)),
            pl.BlockSpec((m_mid, d), lambda i: (0, 0)),
            pl.BlockSpec((d, d), lambda i: (0, 0)),
            pl.BlockSpec((d, d), lambda i: (0, 0)),
            pl.BlockSpec((br, 1), lambda i: (i, 0)),
        ],
        out_specs=[qspec, qspec, qspec, qspec],
        out_shape=[qshape, qshape, qshape, qshape],
    )


def _make_tc_head(n_nodes, d, br):
    grid = (n_nodes // br,)
    h = d // 2

    def body(a0a_ref, a0b_ref, a1a_ref, a1b_ref, hd_ref,
             convB_ref, fusB_ref, cw_ref, tw_ref,
             catTa_ref, catTb_ref, catBa_ref, catBb_ref,
             catb_ref, g_ref, b_ref, out_ref):
        nd = lax.rsqrt(hd_ref[...] + 1.0)
        convB = convB_ref[...]
        fusB = fusB_ref[...]
        cw = cw_ref[...]
        tw = tw_ref[...]
        s0a = (a0a_ref[...] * nd + convB[:, :h]) * cw[:, :h]
        s0b = (a0b_ref[...] * nd + convB[:, h:]) * cw[:, h:]
        s1a = (a1a_ref[...] * nd + fusB[:, :h]) * tw[:, :h]
        s1b = (a1b_ref[...] * nd + fusB[:, h:]) * tw[:, h:]
        pa = jnp.maximum(s0a, 0.0) + jnp.maximum(s1a, 0.0)
        pb = jnp.maximum(s0b, 0.0) + jnp.maximum(s1b, 0.0)
        qa = s0a + s1a
        qb = s0b + s1b
        y = (jnp.dot(pa, catTa_ref[...], preferred_element_type=jnp.float32)
             + jnp.dot(pb, catTb_ref[...], preferred_element_type=jnp.float32)
             + jnp.dot(qa, catBa_ref[...], preferred_element_type=jnp.float32)
             + jnp.dot(qb, catBb_ref[...], preferred_element_type=jnp.float32)
             + catb_ref[...])
        mu = jnp.mean(y, axis=-1, keepdims=True)
        yc = y - mu
        var = jnp.mean(yc * yc, axis=-1, keepdims=True)
        out_ref[...] = yc * lax.rsqrt(var + 1e-5) * g_ref[...] + b_ref[...]

    vec = lambda i: (0, 0)
    qspec = pl.BlockSpec((br, h), lambda i: (i, 0))
    wspec = pl.BlockSpec((h, d), vec)
    return pl.pallas_call(
        body,
        grid=grid,
        in_specs=[
            qspec, qspec, qspec, qspec,
            pl.BlockSpec((br, 1), lambda i: (i, 0)),
            pl.BlockSpec((1, d), vec),
            pl.BlockSpec((1, d), vec),
            pl.BlockSpec((1, d), vec),
            pl.BlockSpec((1, d), vec),
            wspec, wspec, wspec, wspec,
            pl.BlockSpec((1, d), vec),
            pl.BlockSpec((1, d), vec),
            pl.BlockSpec((1, d), vec),
        ],
        out_specs=pl.BlockSpec((br, d), lambda i: (i, 0)),
        out_shape=jax.ShapeDtypeStruct((n_nodes, d), jnp.float32),
    )


def kernel(curr_h, next_h, curr_inc, edge_index, convW, convB, fusW, fusB,
           catW, catB, conv_w, topDown_w, ln_g, ln_b):
    n, d = curr_h.shape
    m = next_h.shape[0]
    e = edge_index.shape[1]
    h = d // 2
    fs = h // _NS

    np_rows = _round_up(n + 1, _NS * _L)
    e_pad = _round_up(e, 4096)

    padn = jnp.full((e_pad - e,), n, jnp.int32)
    src = jnp.concatenate([edge_index[0], padn])
    dst = jnp.concatenate([edge_index[1], padn])

    hist_src, hist_dst = _make_sc_hist(e_pad, np_rows)(src, dst)

    u0a, u0b, u1a, u1b = _make_tc_matmul(n, m, d, np_rows, 512)(
        curr_inc, curr_h, next_h, convW, fusW, hist_src.reshape(-1, 1))

    np_acc = _round_up(n + 1, 128)

    def to_slices(u):
        return (u[:np_acc].reshape(np_acc, _NS, fs)
                .transpose(1, 0, 2).reshape(_NS, np_acc * fs))

    a0a, a0b, a1a, a1b = _make_sc_scatter(e_pad, np_acc, fs)(
        src, dst, to_slices(u0a), to_slices(u0b),
        to_slices(u1a), to_slices(u1b))

    def from_slices(a):
        return (a.reshape(_NS, np_acc, fs)
                .transpose(1, 0, 2).reshape(np_acc, h))

    out = _make_tc_head(n, d, 400)(
        from_slices(a0a), from_slices(a0b),
        from_slices(a1a), from_slices(a1b), hist_dst[:n].reshape(-1, 1),
        convB.reshape(1, -1), fusB.reshape(1, -1),
        conv_w.reshape(1, -1), topDown_w.reshape(1, -1),
        catW[:h], catW[h:d], catW[d:d + h], catW[d + h:],
        catB.reshape(1, -1), ln_g.reshape(1, -1), ln_b.reshape(1, -1))
    return out


# speedup vs baseline: 2.2077x; 2.2077x over previous
import functools

import jax
import jax.numpy as jnp
from jax import lax
from jax.experimental import pallas as pl
from jax.experimental.pallas import tpu as pltpu
from jax.experimental.pallas import tpu_sc as plsc

_NC = 2
_NS = 16
_L = 16


def _round_up(x, m):
    return (x + m - 1) // m * m


def _make_sc_hist(e_pad, np_rows):
    per_tile = e_pad // _NS
    n_vec = per_tile // _L
    seg = np_rows // _NS

    mesh = plsc.VectorSubcoreMesh(core_axis_name="c", subcore_axis_name="s")

    @functools.partial(
        pl.kernel,
        out_type=(
            jax.ShapeDtypeStruct((np_rows,), jnp.float32),
            jax.ShapeDtypeStruct((np_rows,), jnp.float32),
        ),
        mesh=mesh,
        scratch_types=[
            pltpu.VMEM((per_tile,), jnp.int32),
            pltpu.VMEM((np_rows,), jnp.float32),
            pltpu.VMEM((seg,), jnp.float32),
            pltpu.VMEM((seg,), jnp.float32),
            pltpu.VMEM_SHARED((_NS, np_rows), jnp.float32),
        ],
        compiler_params=pltpu.CompilerParams(needs_layout_passes=False),
    )
    def hist_kernel(src_hbm, dst_hbm, out_src, out_dst,
                    idx_v, hist_v, acc_v, seg_v, shared):
        cid = lax.axis_index("c")
        sid = lax.axis_index("s")
        zeros = jnp.zeros((_L,), jnp.float32)
        ones = jnp.ones((_L,), jnp.float32)

        def zero_body(i, _):
            hist_v[pl.ds(i * _L, _L)] = zeros
            return 0
        lax.fori_loop(0, np_rows // _L, zero_body, 0)

        def count_from(ref):
            pltpu.sync_copy(ref.at[pl.ds(sid * per_tile, per_tile)], idx_v)

            def body(k, _):
                iv = idx_v[pl.ds(k * _L, _L)]
                plsc.addupdate_scatter(hist_v, [iv], ones)
                return 0
            lax.fori_loop(0, n_vec, body, 0)

        @pl.when(cid == 0)
        def _():
            count_from(src_hbm)

        @pl.when(cid == 1)
        def _():
            count_from(dst_hbm)

        pltpu.sync_copy(hist_v, shared.at[sid])
        plsc.subcore_barrier()
        pltpu.sync_copy(shared.at[0, pl.ds(sid * seg, seg)], acc_v)

        def red_body(t, _):
            pltpu.sync_copy(shared.at[t, pl.ds(sid * seg, seg)], seg_v)

            def add_body(j, _):
                sl = pl.ds(j * _L, _L)
                acc_v[sl] = acc_v[sl] + seg_v[sl]
                return 0
            lax.fori_loop(0, seg // _L, add_body, 0)
            return 0
        lax.fori_loop(1, _NS, red_body, 0)

        @pl.when(cid == 0)
        def _():
            pltpu.sync_copy(acc_v, out_src.at[pl.ds(sid * seg, seg)])

        @pl.when(cid == 1)
        def _():
            pltpu.sync_copy(acc_v, out_dst.at[pl.ds(sid * seg, seg)])

    return hist_kernel


def _make_sc_scatter(e_rows, np_acc):
    np_rows = np_acc
    rt = e_rows // _NS
    gr = 16
    nbuf = 2
    init_rows = np_rows // _NS
    out_rows = np_rows // _NS

    mesh = plsc.VectorSubcoreMesh(core_axis_name="c", subcore_axis_name="s")

    @functools.partial(
        pl.kernel,
        out_type=(
            jax.ShapeDtypeStruct((np_rows, 128), jnp.float32),
            jax.ShapeDtypeStruct((np_rows, 128), jnp.float32),
        ),
        mesh=mesh,
        scratch_types=[
            pltpu.VMEM((gr, 128), jnp.int32),
            pltpu.VMEM((gr, 128), jnp.int32),
            pltpu.VMEM((128, 128), jnp.float32),
            pltpu.VMEM((128, 128), jnp.float32),
            pltpu.VMEM_SHARED((np_rows, 128), jnp.float32),
            pltpu.SemaphoreType.DMA,
            pltpu.SemaphoreType.DMA,
            pltpu.SemaphoreType.DMA,
            pltpu.SemaphoreType.DMA,
        ],
    )
    def scatter_kernel(src2_hbm, dst2_hbm, u0_hbm, u1_hbm, out0, out1,
                       sidx_v, didx_v, rows_0, rows_1, acc_sh,
                       gsem_0, gsem_1, ssem_0, ssem_1):
        cid = lax.axis_index("c")
        sid = lax.axis_index("s")
        bufs = (rows_0, rows_1)
        gsems = (gsem_0, gsem_1)
        ssems = (ssem_0, ssem_1)

        def run_half(u_hbm, out_hbm):
            pltpu.sync_copy(u_hbm.at[pl.ds(sid * init_rows, init_rows)],
                            acc_sh.at[pl.ds(sid * init_rows, init_rows)])
            plsc.subcore_barrier()

            def group(g, _):
                base = sid * rt + g * gr
                pltpu.sync_copy(src2_hbm.at[pl.ds(base, gr)], sidx_v)
                pltpu.sync_copy(dst2_hbm.at[pl.ds(base, gr)], didx_v)

                gd = [None] * nbuf
                sd = [None] * nbuf
                gd[0] = pltpu.async_copy(u_hbm.at[sidx_v.at[0]], bufs[0],
                                         gsems[0])
                for j in range(gr):
                    b = j % nbuf
                    o = 1 - b
                    gd[b].wait()
                    if j + 1 < gr:
                        if sd[o] is not None:
                            sd[o].wait()
                        gd[o] = pltpu.async_copy(
                            u_hbm.at[sidx_v.at[j + 1]], bufs[o], gsems[o])
                    sd[b] = pltpu.async_copy(
                        bufs[b], acc_sh.at[didx_v.at[j]], ssems[b],
                        add=True)
                sd[0].wait()
                sd[1].wait()
                return 0
            lax.fori_loop(0, rt // gr, group, 0)

            plsc.subcore_barrier()
            pltpu.sync_copy(acc_sh.at[pl.ds(sid * out_rows, out_rows)],
                            out_hbm.at[pl.ds(sid * out_rows, out_rows)])

        @pl.when(cid == 0)
        def _():
            run_half(u0_hbm, out0)

        @pl.when(cid == 1)
        def _():
            run_half(u1_hbm, out1)

    return scatter_kernel


def _make_tc_matmul(n_nodes, m_mid, d, np_rows, br):
    grid = (np_rows // br,)

    def body(ci_ref, ch_ref, nh_ref, convW_ref, fusW_ref, hs_ref,
             u0_ref, u1_ref):
        ns = lax.rsqrt(hs_ref[...] + 1.0)
        ch = ch_ref[...] * ns
        u0_ref[...] = jnp.dot(ch, convW_ref[...],
                              preferred_element_type=jnp.float32)
        fused = jnp.dot(ci_ref[...], nh_ref[...],
                        preferred_element_type=jnp.float32) * ns
        u1_ref[...] = jnp.dot(fused, fusW_ref[...],
                              preferred_element_type=jnp.float32)

    return pl.pallas_call(
        body,
        grid=grid,
        in_specs=[
            pl.BlockSpec((br, m_mid), lambda i: (i, 0)),
            pl.BlockSpec((br, d), lambda i: (i, 0)),
            pl.BlockSpec((m_mid, d), lambda i: (0, 0)),
            pl.BlockSpec((d, d), lambda i: (0, 0)),
            pl.BlockSpec((d, d), lambda i: (0, 0)),
            pl.BlockSpec((br, 1), lambda i: (i, 0)),
        ],
        out_specs=[
            pl.BlockSpec((br, d), lambda i: (i, 0)),
            pl.BlockSpec((br, d), lambda i: (i, 0)),
        ],
        out_shape=[
            jax.ShapeDtypeStruct((np_rows, d), jnp.float32),
            jax.ShapeDtypeStruct((np_rows, d), jnp.float32),
        ],
    )


def _make_tc_head(n_nodes, d, br):
    grid = (n_nodes // br,)

    def body(a0_ref, a1_ref, hd_ref, convB_ref, fusB_ref, cw_ref, tw_ref,
             catT_ref, catB_ref, catb_ref, g_ref, b_ref, out_ref):
        nd = lax.rsqrt(hd_ref[...] + 1.0)
        s1 = (a0_ref[...] * nd + convB_ref[...]) * cw_ref[...]
        s2 = (a1_ref[...] * nd + fusB_ref[...]) * tw_ref[...]
        p = jnp.maximum(s1, 0.0) + jnp.maximum(s2, 0.0)
        q = s1 + s2
        y = (jnp.dot(p, catT_ref[...], preferred_element_type=jnp.float32)
             + jnp.dot(q, catB_ref[...], preferred_element_type=jnp.float32)
             + catb_ref[...])
        mu = jnp.mean(y, axis=-1, keepdims=True)
        yc = y - mu
        var = jnp.mean(yc * yc, axis=-1, keepdims=True)
        out_ref[...] = yc * lax.rsqrt(var + 1e-5) * g_ref[...] + b_ref[...]

    vec = lambda i: (0, 0)
    return pl.pallas_call(
        body,
        grid=grid,
        in_specs=[
            pl.BlockSpec((br, d), lambda i: (i, 0)),
            pl.BlockSpec((br, d), lambda i: (i, 0)),
            pl.BlockSpec((br, 1), lambda i: (i, 0)),
            pl.BlockSpec((1, d), vec),
            pl.BlockSpec((1, d), vec),
            pl.BlockSpec((1, d), vec),
            pl.BlockSpec((1, d), vec),
            pl.BlockSpec((d, d), vec),
            pl.BlockSpec((d, d), vec),
            pl.BlockSpec((1, d), vec),
            pl.BlockSpec((1, d), vec),
            pl.BlockSpec((1, d), vec),
        ],
        out_specs=pl.BlockSpec((br, d), lambda i: (i, 0)),
        out_shape=jax.ShapeDtypeStruct((n_nodes, d), jnp.float32),
    )


def kernel(curr_h, next_h, curr_inc, edge_index, convW, convB, fusW, fusB,
           catW, catB, conv_w, topDown_w, ln_g, ln_b):
    n, d = curr_h.shape
    m = next_h.shape[0]
    e = edge_index.shape[1]

    np_rows = _round_up(n + 1, _NS * _L)
    e_rows = _round_up(e, _NS * 8 * 128) // 128
    e_pad = e_rows * 128

    padn = jnp.full((e_pad - e,), n, jnp.int32)
    src = jnp.concatenate([edge_index[0], padn])
    dst = jnp.concatenate([edge_index[1], padn])
    src2 = src.reshape(e_rows, 128)
    dst2 = dst.reshape(e_rows, 128)

    hist_src, hist_dst = _make_sc_hist(e_pad, np_rows)(src, dst)

    u0, u1 = _make_tc_matmul(n, m, d, np_rows, 512)(
        curr_inc, curr_h, next_h, convW, fusW, hist_src.reshape(-1, 1))

    np_acc = _round_up(n + 1, 128)
    agg0, agg1 = _make_sc_scatter(e_rows, np_acc)(src2, dst2, u0, u1)

    out = _make_tc_head(n, d, 400)(
        agg0, agg1, hist_dst[:n].reshape(-1, 1),
        convB.reshape(1, -1), fusB.reshape(1, -1),
        conv_w.reshape(1, -1), topDown_w.reshape(1, -1),
        catW[:d], catW[d:], catB.reshape(1, -1),
        ln_g.reshape(1, -1), ln_b.reshape(1, -1))
    return out
